# baseline (speedup 1.0000x reference)
import functools

import jax
import jax.numpy as jnp
import numpy as np
from jax import lax
from jax.experimental import pallas as pl
from jax.experimental.pallas import tpu as pltpu
from jax.experimental.pallas import tpu_sc as plsc

N = 10000
E = 320000
D = 128

NC = 2
NS = 16
NW = NC * NS
L = 16

NP = 10240
EP = 327680
EPW = EP // NW
B = 128
NB = EPW // B
RPT = NP // NS
CH = 64

_blk = np.concatenate([np.arange(0, 32, 2), np.arange(1, 32, 2)])
_PERM = np.concatenate([32 * i + _blk for i in range(4)])

_mesh = plsc.VectorSubcoreMesh(core_axis_name="c", subcore_axis_name="s")
_sc_params = pltpu.CompilerParams(
    needs_layout_passes=False, use_tc_tiling_on_sc=False)


_RB = 1024


def _precompute_body(x_ref, wn_ref, ws_ref, b_ref, y_ref, c_ref):
    x = x_ref[...]
    y_ref[...] = jnp.dot(x, wn_ref[...], preferred_element_type=jnp.float32)
    c_ref[...] = (
        jnp.dot(x, ws_ref[...], preferred_element_type=jnp.float32) + b_ref[...])


_precompute = pl.pallas_call(
    _precompute_body,
    grid=(NP // _RB,),
    in_specs=[
        pl.BlockSpec((_RB, D), lambda i: (i, 0)),
        pl.BlockSpec((D, D), lambda i: (0, 0)),
        pl.BlockSpec((D, D), lambda i: (0, 0)),
        pl.BlockSpec((1, D), lambda i: (0, 0)),
    ],
    out_specs=[
        pl.BlockSpec((_RB, D), lambda i: (i, 0)),
        pl.BlockSpec((_RB, D), lambda i: (i, 0)),
    ],
    out_shape=[
        jax.ShapeDtypeStruct((NP, D), jnp.float32),
        jax.ShapeDtypeStruct((NP, D), jnp.float32),
    ],
)


B1 = 64
NB1 = EPW // B1
NBH1 = NB1 // 2


@functools.partial(
    pl.kernel,
    out_type=jax.ShapeDtypeStruct((NC, NP, D), jnp.float32),
    mesh=_mesh,
    scratch_types=[
        pltpu.VMEM((NBH1, B1), jnp.int32),
        pltpu.VMEM((NBH1, B1), jnp.int32),
        pltpu.VMEM((B1, D // 2), jnp.int32),
        pltpu.VMEM((B1, D // 2), jnp.int32),
        pltpu.VMEM((B1, D), jnp.float32),
        pltpu.VMEM((B1, D), jnp.float32),
        pltpu.VMEM_SHARED((NP, D), jnp.float32),
        pltpu.SemaphoreType.DMA,
        pltpu.SemaphoreType.DMA,
        pltpu.SemaphoreType.DMA,
        pltpu.SemaphoreType.DMA,
    ],
    compiler_params=_sc_params,
)
def _segment_sum(src_hbm, tgt_hbm, y32_hbm, cperm_hbm, zeros_hbm, out_hbm,
                 idx_s, idx_t, bf0, bf1, f0, f1, agg_sh,
                 gsem0, gsem1, ssem0, ssem1):
    c = lax.axis_index("c")
    s = lax.axis_index("s")
    wid = c * NS + s
    bfs = (bf0, bf1)
    f32s = (f0, f1)
    gsems = (gsem0, gsem1)
    ssems = (ssem0, ssem1)

    @pl.when(c == 0)
    def _():
        pltpu.sync_copy(cperm_hbm.at[pl.ds(s * RPT, RPT)],
                        agg_sh.at[pl.ds(s * RPT, RPT)])

    @pl.when(c != 0)
    def _():
        pltpu.sync_copy(zeros_hbm, agg_sh.at[pl.ds(s * RPT, RPT)])

    plsc.subcore_barrier()

    def process(i, b):
        pltpu.make_async_copy(y32_hbm.at[pl.ds(0, B1)], bfs[b], gsems[b]).wait()
        bfb = bfs[b]
        fb = f32s[b]

        @plsc.parallel_loop(0, B1, step=1, unroll=8)
        def conv(e):
            for k in range(D // 32):
                v = plsc.bitcast(bfb[e, pl.ds(k * L, L)], jnp.bfloat16)
                va, vb = plsc.unpack(v, format=plsc.PackFormat.INTERLEAVED)
                fb[e, pl.ds(k * 32, L)] = va
                fb[e, pl.ds(k * 32 + L, L)] = vb

        pltpu.sync_copy(fb, agg_sh.at[idx_t.at[i]], add=True)
        inext = jnp.minimum(i + 2, NBH1 - 1)
        pltpu.async_copy(y32_hbm.at[idx_s.at[inext]], bfs[b], gsems[b])

    for half in range(2):
        pltpu.sync_copy(src_hbm.at[wid, pl.ds(half * NBH1, NBH1)], idx_s)
        pltpu.sync_copy(tgt_hbm.at[wid, pl.ds(half * NBH1, NBH1)], idx_t)

        for b in range(2):
            pltpu.async_copy(y32_hbm.at[idx_s.at[b]], bfs[b], gsems[b])

        def it_body(it, carry):
            process(it * 2, 0)
            process(it * 2 + 1, 1)
            return carry

        lax.fori_loop(0, NBH1 // 2, it_body, 0)
        for b in range(2):
            pltpu.make_async_copy(y32_hbm.at[pl.ds(0, B1)], bfs[b], gsems[b]).wait()
    plsc.subcore_barrier()

    pltpu.sync_copy(agg_sh.at[pl.ds(s * RPT, RPT)],
                    out_hbm.at[c, pl.ds(s * RPT, RPT)])


@functools.partial(
    pl.kernel,
    out_type=(jax.ShapeDtypeStruct((EP,), jnp.float32),
              jax.ShapeDtypeStruct((NP, D // 2), jnp.int32)),
    mesh=_mesh,
    scratch_types=[
        pltpu.VMEM((NB, B), jnp.int32),
        pltpu.VMEM((NB, B), jnp.int32),
        pltpu.VMEM((CH, D), jnp.float32),
        pltpu.VMEM((CH, D), jnp.float32),
        pltpu.VMEM((CH, D // 2), jnp.int32),
        pltpu.VMEM((B, D // 2), jnp.int32),
        pltpu.VMEM((B, D // 2), jnp.int32),
        pltpu.VMEM((B, D // 2), jnp.int32),
        pltpu.VMEM((B, D // 2), jnp.int32),
        pltpu.VMEM((B,), jnp.float32),
        pltpu.VMEM_SHARED((NP, D // 2), jnp.int32),
        pltpu.SemaphoreType.DMA,
        pltpu.SemaphoreType.DMA,
    ],
    compiler_params=_sc_params,
)
def _edge_dots(src_hbm, tgt_hbm, z_hbm, out_hbm, hd_hbm,
               idx_s, idx_t, zbuf0, zbuf1, hbuf,
               rs0, rs1, rt0, rt1, out_v, h_sh, sem0, sem1):
    c = lax.axis_index("c")
    s = lax.axis_index("s")
    wid = c * NS + s
    rows_s = (rs0, rs1)
    rows_t = (rt0, rt1)
    sems = (sem0, sem1)

    def ch_body(ch, carry):
        r0 = s * RPT + ch * CH
        pltpu.sync_copy(z_hbm.at[0, pl.ds(r0, CH)], zbuf0)
        pltpu.sync_copy(z_hbm.at[1, pl.ds(r0, CH)], zbuf1)

        @plsc.parallel_loop(0, CH, step=1, unroll=8)
        def row_body(r):
            for k in range(D // 32):
                za = zbuf0[r, pl.ds(k * 32, L)] + zbuf1[r, pl.ds(k * 32, L)]
                zb = zbuf0[r, pl.ds(k * 32 + L, L)] + zbuf1[r, pl.ds(k * 32 + L, L)]
                ha = jnp.maximum(za, 0.0)
                hb = jnp.maximum(zb, 0.0)
                packed = plsc.pack(ha, hb, format=plsc.PackFormat.INTERLEAVED)
                hbuf[r, pl.ds(k * L, L)] = plsc.bitcast(packed, jnp.int32)
        pltpu.sync_copy(hbuf, h_sh.at[pl.ds(r0, CH)])
        pltpu.sync_copy(hbuf, hd_hbm.at[pl.ds(r0, CH)])
        return carry

    lax.fori_loop(0, RPT // CH, ch_body, 0)
    pltpu.sync_copy(src_hbm.at[wid], idx_s)
    pltpu.sync_copy(tgt_hbm.at[wid], idx_t)
    plsc.subcore_barrier()

    ebase = wid * EPW

    for b in range(2):
        pltpu.async_copy(h_sh.at[idx_s.at[b]], rows_s[b], sems[b])
        pltpu.async_copy(h_sh.at[idx_t.at[b]], rows_t[b], sems[b])

    def it_body(it, carry):
        for b in range(2):
            i = it * 2 + b
            pltpu.make_async_copy(hd_hbm.at[pl.ds(0, B)], rows_s[b], sems[b]).wait()
            pltpu.make_async_copy(hd_hbm.at[pl.ds(0, B)], rows_t[b], sems[b]).wait()
            rs, rt = rows_s[b], rows_t[b]

            @plsc.parallel_loop(0, B // L, step=1, unroll=2)
            def g_body(g):
                res = jnp.zeros((L,), jnp.float32)
                for j in range(L):
                    e = g * L + j
                    acc = None
                    for k in range(D // 32):
                        vs = plsc.bitcast(rs[e, pl.ds(k * L, L)], jnp.bfloat16)
                        vt = plsc.bitcast(rt[e, pl.ds(k * L, L)], jnp.bfloat16)
                        pa, pb = plsc.unpack(vs * vt,
                                             format=plsc.PackFormat.INTERLEAVED)
                        p = pa + pb
                        acc = p if acc is None else acc + p
                    tot = jnp.sum(acc)
                    onehot = (lax.iota(jnp.int32, L) == j).astype(jnp.float32)
                    res = res + tot * onehot
                out_v[pl.ds(g * L, L)] = res
            pltpu.sync_copy(out_v, out_hbm.at[pl.ds(ebase + i * B, B)])
            inext = jnp.minimum(i + 2, NB - 1)
            pltpu.async_copy(h_sh.at[idx_s.at[inext]], rows_s[b], sems[b])
            pltpu.async_copy(h_sh.at[idx_t.at[inext]], rows_t[b], sems[b])
        return carry

    lax.fori_loop(0, NB // 2, it_body, 0)
    for b in range(2):
        pltpu.make_async_copy(hd_hbm.at[pl.ds(0, B)], rows_s[b], sems[b]).wait()
        pltpu.make_async_copy(hd_hbm.at[pl.ds(0, B)], rows_t[b], sems[b]).wait()


def kernel(x, edge_index, W_neigh, W_self, b):
    src = edge_index[0]
    tgt = edge_index[1]
    npad = EP - E
    pad_ids = jnp.arange(npad, dtype=jnp.int32)
    src_f = jnp.concatenate([src, pad_ids % N])
    tgt_f = jnp.concatenate([tgt, N + (pad_ids % (NP - N))])
    src_p = src_f.reshape(NW, NB, B)
    tgt_p = tgt_f.reshape(NW, NB, B)
    src_p1 = src_f.reshape(NW, NB1, B1)
    tgt_p1 = tgt_f.reshape(NW, NB1, B1)
    xp = jnp.pad(x, ((0, NP - N), (0, 0)))
    zeros = jnp.zeros((RPT, D), jnp.float32)

    perm = jnp.asarray(_PERM)
    y, cperm = _precompute(xp, W_neigh, W_self[:, perm], b[perm].reshape(1, D))
    y32 = lax.bitcast_convert_type(
        y.astype(jnp.bfloat16).reshape(NP, D // 2, 2), jnp.int32)

    z2 = _segment_sum(src_p1, tgt_p1, y32, cperm, zeros)
    scores, _ = _edge_dots(src_p, tgt_p, z2)
    return scores.reshape(EP)[:E]


# speedup vs baseline: 1.1295x; 1.1295x over previous
import functools

import jax
import jax.numpy as jnp
from jax import lax
from jax.experimental import pallas as pl
from jax.experimental.pallas import tpu as pltpu
from jax.experimental.pallas import tpu_sc as plsc

N = 10000
E = 320000
D = 128

NC = 2
NS = 16
NW = NC * NS
L = 16

NP = 10240
EP = 327680
EPW = EP // NW
B = 128
NB = EPW // B
RPT = NP // NS

_mesh = plsc.VectorSubcoreMesh(core_axis_name="c", subcore_axis_name="s")


@functools.partial(
    pl.kernel,
    out_type=jax.ShapeDtypeStruct((NC, NP, D), jnp.float32),
    mesh=_mesh,
    scratch_types=[
        pltpu.VMEM((NB // 2, B), jnp.int32),
        pltpu.VMEM((NB // 2, B), jnp.int32),
        pltpu.VMEM((B, D), jnp.float32),
        pltpu.VMEM((B, D), jnp.float32),
        pltpu.VMEM_SHARED((NP, D), jnp.float32),
        pltpu.SemaphoreType.DMA,
        pltpu.SemaphoreType.DMA,
    ],
    compiler_params=pltpu.CompilerParams(needs_layout_passes=False),
)
def _segment_sum(src_hbm, tgt_hbm, x_hbm, zeros_hbm, out_hbm,
                 idx_s, idx_t, rows0, rows1, agg_sh, sem0, sem1):
    c = lax.axis_index("c")
    s = lax.axis_index("s")
    wid = c * NS + s
    rows = (rows0, rows1)
    sems = (sem0, sem1)
    NBH = NB // 2

    pltpu.sync_copy(zeros_hbm, agg_sh.at[pl.ds(s * RPT, RPT)])
    plsc.subcore_barrier()

    for half in range(2):
        pltpu.sync_copy(src_hbm.at[wid, pl.ds(half * NBH, NBH)], idx_s)
        pltpu.sync_copy(tgt_hbm.at[wid, pl.ds(half * NBH, NBH)], idx_t)

        for b in range(2):
            pltpu.async_copy(x_hbm.at[idx_s.at[b]], rows[b], sems[b])

        def it_body(it, carry):
            for b in range(2):
                i = it * 2 + b
                pltpu.make_async_copy(x_hbm.at[pl.ds(0, B)], rows[b], sems[b]).wait()
                pltpu.sync_copy(rows[b], agg_sh.at[idx_t.at[i]], add=True)
                inext = jnp.minimum(i + 2, NBH - 1)
                pltpu.async_copy(x_hbm.at[idx_s.at[inext]], rows[b], sems[b])
            return carry

        lax.fori_loop(0, NBH // 2, it_body, 0)
        for b in range(2):
            pltpu.make_async_copy(x_hbm.at[pl.ds(0, B)], rows[b], sems[b]).wait()
    plsc.subcore_barrier()

    pltpu.sync_copy(agg_sh.at[pl.ds(s * RPT, RPT)],
                    out_hbm.at[c, pl.ds(s * RPT, RPT)])


_RB = 1024


def _encoder_body(agg_ref, x_ref, wn_ref, ws_ref, b_ref, o_ref):
    agg = agg_ref[0] + agg_ref[1]
    acc = jnp.dot(agg, wn_ref[...], preferred_element_type=jnp.float32)
    acc += jnp.dot(x_ref[...], ws_ref[...], preferred_element_type=jnp.float32)
    acc += b_ref[...]
    o_ref[...] = jnp.maximum(acc, 0.0).astype(jnp.bfloat16)


_encoder = pl.pallas_call(
    _encoder_body,
    grid=(NP // _RB,),
    in_specs=[
        pl.BlockSpec((NC, _RB, D), lambda i: (0, i, 0)),
        pl.BlockSpec((_RB, D), lambda i: (i, 0)),
        pl.BlockSpec((D, D), lambda i: (0, 0)),
        pl.BlockSpec((D, D), lambda i: (0, 0)),
        pl.BlockSpec((1, D), lambda i: (0, 0)),
    ],
    out_specs=pl.BlockSpec((_RB, D), lambda i: (i, 0)),
    out_shape=jax.ShapeDtypeStruct((NP, D), jnp.bfloat16),
)


@functools.partial(
    pl.kernel,
    out_type=jax.ShapeDtypeStruct((EP,), jnp.float32),
    mesh=_mesh,
    scratch_types=[
        pltpu.VMEM((NB, B), jnp.int32),
        pltpu.VMEM((NB, B), jnp.int32),
        pltpu.VMEM((B, D // 2), jnp.int32),
        pltpu.VMEM((B, D // 2), jnp.int32),
        pltpu.VMEM((B, D // 2), jnp.int32),
        pltpu.VMEM((B, D // 2), jnp.int32),
        pltpu.VMEM((B,), jnp.float32),
        pltpu.SemaphoreType.DMA,
        pltpu.SemaphoreType.DMA,
    ],
    compiler_params=pltpu.CompilerParams(
        needs_layout_passes=False, use_tc_tiling_on_sc=False),
)
def _edge_dots(src_hbm, tgt_hbm, h_hbm, out_hbm,
               idx_s, idx_t, rs0, rs1, rt0, rt1, out_v, sem0, sem1):
    c = lax.axis_index("c")
    s = lax.axis_index("s")
    wid = c * NS + s
    rows_s = (rs0, rs1)
    rows_t = (rt0, rt1)
    sems = (sem0, sem1)

    pltpu.sync_copy(src_hbm.at[wid], idx_s)
    pltpu.sync_copy(tgt_hbm.at[wid], idx_t)

    ebase = wid * EPW

    for b in range(2):
        pltpu.async_copy(h_hbm.at[idx_s.at[b]], rows_s[b], sems[b])
        pltpu.async_copy(h_hbm.at[idx_t.at[b]], rows_t[b], sems[b])

    def it_body(it, carry):
        for b in range(2):
            i = it * 2 + b
            pltpu.make_async_copy(h_hbm.at[pl.ds(0, B)], rows_s[b], sems[b]).wait()
            pltpu.make_async_copy(h_hbm.at[pl.ds(0, B)], rows_t[b], sems[b]).wait()
            rs, rt = rows_s[b], rows_t[b]

            @plsc.parallel_loop(0, B // L, step=1, unroll=2)
            def g_body(g):
                res = jnp.zeros((L,), jnp.float32)
                for j in range(L):
                    e = g * L + j
                    acc = None
                    for k in range(D // 32):
                        vs = plsc.bitcast(rs[e, pl.ds(k * L, L)], jnp.bfloat16)
                        vt = plsc.bitcast(rt[e, pl.ds(k * L, L)], jnp.bfloat16)
                        pa, pb = plsc.unpack(vs * vt,
                                             format=plsc.PackFormat.INTERLEAVED)
                        p = pa + pb
                        acc = p if acc is None else acc + p
                    tot = jnp.sum(acc)
                    onehot = (lax.iota(jnp.int32, L) == j).astype(jnp.float32)
                    res = res + tot * onehot
                out_v[pl.ds(g * L, L)] = res

            pltpu.sync_copy(out_v, out_hbm.at[pl.ds(ebase + i * B, B)])
            inext = jnp.minimum(i + 2, NB - 1)
            pltpu.async_copy(h_hbm.at[idx_s.at[inext]], rows_s[b], sems[b])
            pltpu.async_copy(h_hbm.at[idx_t.at[inext]], rows_t[b], sems[b])
        return carry

    lax.fori_loop(0, NB // 2, it_body, 0)
    for b in range(2):
        pltpu.make_async_copy(h_hbm.at[pl.ds(0, B)], rows_s[b], sems[b]).wait()
        pltpu.make_async_copy(h_hbm.at[pl.ds(0, B)], rows_t[b], sems[b]).wait()


def kernel(x, edge_index, W_neigh, W_self, b):
    src = edge_index[0]
    tgt = edge_index[1]
    npad = EP - E
    pad_ids = jnp.arange(npad, dtype=jnp.int32)
    src_p = jnp.concatenate([src, pad_ids % N]).reshape(NW, NB, B)
    tgt_p = jnp.concatenate([tgt, N + (pad_ids % (NP - N))]).reshape(NW, NB, B)
    xp = jnp.pad(x, ((0, NP - N), (0, 0)))
    zeros = jnp.zeros((RPT, D), jnp.float32)

    agg2 = _segment_sum(src_p, tgt_p, xp, zeros)
    h = _encoder(agg2, xp, W_neigh, W_self, b.reshape(1, D))
    h32 = lax.bitcast_convert_type(h.reshape(NP, D // 2, 2), jnp.int32)
    scores = _edge_dots(src_p, tgt_p, h32)
    return scores.reshape(EP)[:E]
